# Initial kernel scaffold; baseline (speedup 1.0000x reference)
import jax
import jax.numpy as jnp
from jax.experimental import pallas as pl


def kernel(z_demand, z_measurement, edge_label_index, W1, b1, W2, b2):
    raise NotImplementedError("write your pallas kernel here")


# speedup vs baseline: 41.7992x; 41.7992x over previous
import functools

import jax
import jax.numpy as jnp
from jax import lax
from jax.experimental import pallas as pl
from jax.experimental.pallas import tpu as pltpu
from jax.experimental.pallas import tpu_sc as plsc

_N_NODES = 100000
_N_EDGES = 6400000
_HID = 8

_NC = 2
_NS = 16
_NW = _NC * _NS
_EPW = _N_EDGES // _NW
_C = 1600
_G = 64
_NG = _C // _G
_NCHUNK = _EPW // _C
_NGRP = _C // 16


def _precompute_body(zd_ref, zm_ref, w1a_ref, w1b_ref, b1_ref, ad_ref, bm_ref):
    dn = (((1,), (1,)), ((), ()))
    ad_ref[...] = lax.dot_general(zd_ref[...], w1a_ref[...], dn,
                                  preferred_element_type=jnp.float32)
    bm_ref[...] = lax.dot_general(zm_ref[...], w1b_ref[...], dn,
                                  preferred_element_type=jnp.float32) + b1_ref[...]


def _precompute(zd, zm, w1a, w1b, b1):
    blk = 2000
    grid = _N_NODES // blk
    return pl.pallas_call(
        _precompute_body,
        grid=(grid,),
        in_specs=[
            pl.BlockSpec((blk, _HID), lambda i: (i, 0)),
            pl.BlockSpec((blk, _HID), lambda i: (i, 0)),
            pl.BlockSpec((_HID, _HID), lambda i: (0, 0)),
            pl.BlockSpec((_HID, _HID), lambda i: (0, 0)),
            pl.BlockSpec((1, _HID), lambda i: (0, 0)),
        ],
        out_specs=[
            pl.BlockSpec((blk, _HID), lambda i: (i, 0)),
            pl.BlockSpec((blk, _HID), lambda i: (i, 0)),
        ],
        out_shape=[
            jax.ShapeDtypeStruct((_N_NODES, _HID), jnp.float32),
            jax.ShapeDtypeStruct((_N_NODES, _HID), jnp.float32),
        ],
    )(zd, zm, w1a, w1b, b1)


def _edge_body(ad_hbm, bm_hbm, row_hbm, col_hbm, aux_hbm, out_hbm,
               ridx_v, cidx_v, za_v, zb_v, out_v, aux_v, gsem):
    wid = lax.axis_index("s") * _NC + lax.axis_index("c")
    tbase = wid * _EPW

    pltpu.sync_copy(aux_hbm, aux_v)
    iota = lax.iota(jnp.int32, 16)
    w2f = [plsc.load_gather(aux_v, [jnp.full((16,), 1 + f, jnp.int32)])
           for f in range(_HID)]
    vb2 = plsc.load_gather(aux_v, [jnp.full((16,), 1 + _HID, jnp.int32)])

    def chunk_body(i, carry):
        gbase = tbase + i * _C
        pltpu.sync_copy(row_hbm.at[pl.ds(gbase, _C)], ridx_v)
        pltpu.sync_copy(col_hbm.at[pl.ds(gbase, _C)], cidx_v)
        copies = []
        for j in range(_NG):
            sl = pl.ds(j * _G, _G)
            copies.append(pltpu.async_copy(ad_hbm.at[ridx_v.at[sl]],
                                           za_v.at[sl], gsem))
            copies.append(pltpu.async_copy(bm_hbm.at[cidx_v.at[sl]],
                                           zb_v.at[sl], gsem))
        for cp in copies:
            cp.wait()

        def grp_body(g, carry2):
            base = g * 16
            lanes = base + iota
            acc = vb2
            for f in range(_HID):
                fi = jnp.full((16,), f, jnp.int32)
                a = plsc.load_gather(za_v, [lanes, fi])
                b = plsc.load_gather(zb_v, [lanes, fi])
                acc = acc + w2f[f] * jnp.maximum(a + b, 0.0)
            out_v[pl.ds(base, 16)] = acc
            return carry2

        lax.fori_loop(0, _NGRP, grp_body, 0)
        pltpu.sync_copy(out_v, out_hbm.at[pl.ds(gbase, _C)])
        return carry

    lax.fori_loop(0, _NCHUNK, chunk_body, 0)


@functools.partial(jax.jit, static_argnums=())
def kernel(z_demand, z_measurement, edge_label_index, W1, b1, W2, b2):
    w1a = W1[:, :_HID]
    w1b = W1[:, _HID:]
    ad, bm = _precompute(z_demand, z_measurement, w1a, w1b, b1.reshape(1, _HID))

    eli = edge_label_index.astype(jnp.int32)
    row = eli[0]
    col = eli[1]
    aux = jnp.concatenate([jnp.zeros((1,), jnp.float32), W2.reshape(-1),
                           b2.reshape(-1),
                           jnp.zeros((16 - _HID - 2,), jnp.float32)])

    edge_kernel = pl.kernel(
        _edge_body,
        out_type=jax.ShapeDtypeStruct((_N_EDGES,), jnp.float32),
        mesh=plsc.VectorSubcoreMesh(core_axis_name="c", subcore_axis_name="s"),
        compiler_params=pltpu.CompilerParams(needs_layout_passes=False,
                                             use_tc_tiling_on_sc=False),
        scratch_types=[
            pltpu.VMEM((_C,), jnp.int32),
            pltpu.VMEM((_C,), jnp.int32),
            pltpu.VMEM((_C, _HID), jnp.float32),
            pltpu.VMEM((_C, _HID), jnp.float32),
            pltpu.VMEM((_C,), jnp.float32),
            pltpu.VMEM((16,), jnp.float32),
            pltpu.SemaphoreType.DMA,
        ],
    )
    return edge_kernel(ad, bm, row, col, aux)


# speedup vs baseline: 74.5089x; 1.7825x over previous
import functools

import jax
import jax.numpy as jnp
from jax import lax
from jax.experimental import pallas as pl
from jax.experimental.pallas import tpu as pltpu
from jax.experimental.pallas import tpu_sc as plsc

_N_NODES = 100000
_N_EDGES = 6400000
_HID = 8

_NC = 2
_NS = 16
_NW = _NC * _NS
_EPW = _N_EDGES // _NW
_C = 2000
_G = 80
_NG = _C // _G
_NCHUNK = _EPW // _C
_NGRP = _C // 16
_UNROLL = 2


def _precompute_body(zd_ref, zm_ref, w1a_ref, w1b_ref, b1_ref, ad_ref, bm_ref):
    dn = (((1,), (1,)), ((), ()))
    ad_ref[...] = lax.dot_general(zd_ref[...], w1a_ref[...], dn,
                                  preferred_element_type=jnp.float32)
    bm_ref[...] = lax.dot_general(zm_ref[...], w1b_ref[...], dn,
                                  preferred_element_type=jnp.float32) + b1_ref[...]


def _precompute(zd, zm, w1a, w1b, b1):
    blk = 2000
    grid = _N_NODES // blk
    return pl.pallas_call(
        _precompute_body,
        grid=(grid,),
        in_specs=[
            pl.BlockSpec((blk, _HID), lambda i: (i, 0)),
            pl.BlockSpec((blk, _HID), lambda i: (i, 0)),
            pl.BlockSpec((_HID, _HID), lambda i: (0, 0)),
            pl.BlockSpec((_HID, _HID), lambda i: (0, 0)),
            pl.BlockSpec((1, _HID), lambda i: (0, 0)),
        ],
        out_specs=[
            pl.BlockSpec((blk, _HID), lambda i: (i, 0)),
            pl.BlockSpec((blk, _HID), lambda i: (i, 0)),
        ],
        out_shape=[
            jax.ShapeDtypeStruct((_N_NODES, _HID), jnp.float32),
            jax.ShapeDtypeStruct((_N_NODES, _HID), jnp.float32),
        ],
    )(zd, zm, w1a, w1b, b1)


def _edge_body(ad_hbm, bm_hbm, eli_hbm, aux_hbm, out_hbm,
               r0, c0, a0, b0, o0, r1, c1, a1, b1_, o1, aux_v,
               gs0, gs1, is0, is1, os0, os1):
    wid = lax.axis_index("s") * _NC + lax.axis_index("c")
    tbase = wid * _EPW
    B0 = (r0, c0, a0, b0, o0, gs0, is0, os0)
    B1 = (r1, c1, a1, b1_, o1, gs1, is1, os1)

    pltpu.sync_copy(aux_hbm, aux_v)
    iota = lax.iota(jnp.int32, 16)
    w2f = [plsc.load_gather(aux_v, [jnp.full((16,), 1 + f, jnp.int32)])
           for f in range(_HID)]
    vb2 = plsc.load_gather(aux_v, [jnp.full((16,), 1 + _HID, jnp.int32)])

    def idx_issue(buf, i):
        r, c, _, _, _, _, isem, _ = buf
        gbase = tbase + i * _C
        pltpu.async_copy(eli_hbm.at[0, pl.ds(gbase, _C)], r, isem)
        pltpu.async_copy(eli_hbm.at[1, pl.ds(gbase, _C)], c, isem)

    def idx_wait(buf):
        r, c, _, _, _, _, isem, _ = buf
        pltpu.make_async_copy(eli_hbm.at[0, pl.ds(0, _C)], r, isem).wait()
        pltpu.make_async_copy(eli_hbm.at[1, pl.ds(0, _C)], c, isem).wait()

    def fire(buf):
        r, c, za, zb, _, gsem, _, _ = buf
        for j in range(_NG):
            sl = pl.ds(j * _G, _G)
            pltpu.async_copy(ad_hbm.at[r.at[sl]], za.at[sl], gsem)
            pltpu.async_copy(bm_hbm.at[c.at[sl]], zb.at[sl], gsem)

    def drain(buf):
        _, _, za, zb, _, gsem, _, _ = buf
        pltpu.make_async_copy(ad_hbm.at[pl.ds(0, _C)], za, gsem).wait()
        pltpu.make_async_copy(bm_hbm.at[pl.ds(0, _C)], zb, gsem).wait()

    def compute_store(buf, i):
        _, _, za, zb, o, _, _, osem = buf
        gbase = tbase + i * _C

        @plsc.parallel_loop(0, _NGRP, 1, unroll=_UNROLL)
        def grp_body(g):
            base = g * 16
            lanes = base + iota
            acc = vb2
            for f in range(_HID):
                fi = jnp.full((16,), f, jnp.int32)
                a = plsc.load_gather(za, [lanes, fi])
                b = plsc.load_gather(zb, [lanes, fi])
                acc = acc + w2f[f] * jnp.maximum(a + b, 0.0)
            o[pl.ds(base, 16)] = acc

        pltpu.async_copy(o, out_hbm.at[pl.ds(gbase, _C)], osem)

    def out_wait(buf):
        _, _, _, _, o, _, _, osem = buf
        pltpu.make_async_copy(o, out_hbm.at[pl.ds(0, _C)], osem).wait()

    idx_issue(B0, 0)
    idx_wait(B0)
    fire(B0)
    idx_issue(B1, 1)

    idx_wait(B1); fire(B1)
    drain(B0); idx_issue(B0, 2)
    compute_store(B0, 0)
    idx_wait(B0); fire(B0)
    drain(B1); idx_issue(B1, 3)
    compute_store(B1, 1)

    def pair_body(p, carry):
        i = 2 * p
        idx_wait(B1); fire(B1)
        drain(B0); idx_issue(B0, i + 2)
        out_wait(B0)
        compute_store(B0, i)
        idx_wait(B0); fire(B0)
        drain(B1); idx_issue(B1, i + 3)
        out_wait(B1)
        compute_store(B1, i + 1)
        return carry

    lax.fori_loop(1, _NCHUNK // 2 - 1, pair_body, 0)

    i = _NCHUNK - 2
    idx_wait(B1); fire(B1)
    drain(B0)
    out_wait(B0)
    compute_store(B0, i)
    drain(B1)
    out_wait(B1)
    compute_store(B1, i + 1)
    out_wait(B0)
    out_wait(B1)


@functools.partial(jax.jit, static_argnums=())
def kernel(z_demand, z_measurement, edge_label_index, W1, b1, W2, b2):
    w1a = W1[:, :_HID]
    w1b = W1[:, _HID:]
    ad, bm = _precompute(z_demand, z_measurement, w1a, w1b, b1.reshape(1, _HID))

    eli = edge_label_index.astype(jnp.int32)
    aux = jnp.concatenate([jnp.zeros((1,), jnp.float32), W2.reshape(-1),
                           b2.reshape(-1),
                           jnp.zeros((16 - _HID - 2,), jnp.float32)])

    edge_kernel = pl.kernel(
        _edge_body,
        out_type=jax.ShapeDtypeStruct((_N_EDGES,), jnp.float32),
        mesh=plsc.VectorSubcoreMesh(core_axis_name="c", subcore_axis_name="s"),
        compiler_params=pltpu.CompilerParams(needs_layout_passes=False,
                                             use_tc_tiling_on_sc=False),
        scratch_types=[
            pltpu.VMEM((_C,), jnp.int32),
            pltpu.VMEM((_C,), jnp.int32),
            pltpu.VMEM((_C, _HID), jnp.float32),
            pltpu.VMEM((_C, _HID), jnp.float32),
            pltpu.VMEM((_C,), jnp.float32),
            pltpu.VMEM((_C,), jnp.int32),
            pltpu.VMEM((_C,), jnp.int32),
            pltpu.VMEM((_C, _HID), jnp.float32),
            pltpu.VMEM((_C, _HID), jnp.float32),
            pltpu.VMEM((_C,), jnp.float32),
            pltpu.VMEM((16,), jnp.float32),
            pltpu.SemaphoreType.DMA,
            pltpu.SemaphoreType.DMA,
            pltpu.SemaphoreType.DMA,
            pltpu.SemaphoreType.DMA,
            pltpu.SemaphoreType.DMA,
            pltpu.SemaphoreType.DMA,
        ],
    )
    return edge_kernel(ad, bm, eli, aux)


# speedup vs baseline: 82.5713x; 1.1082x over previous
import functools

import jax
import jax.numpy as jnp
from jax import lax
from jax.experimental import pallas as pl
from jax.experimental.pallas import tpu as pltpu
from jax.experimental.pallas import tpu_sc as plsc

_N_NODES = 100000
_N_EDGES = 6400000
_HID = 8

_NC = 2
_NS = 16
_NW = _NC * _NS
_EPW = _N_EDGES // _NW
_C = 2000
_G = 80
_NG = _C // _G
_NCHUNK = _EPW // _C
_NGRP = _C // 16
_UNROLL = 4


_PROWS = _N_NODES * _HID // 128


def _precompute_body(zd2_ref, zm2_ref, wa_ref, wb_ref, b1t_ref, ad2_ref, bm2_ref):
    ad2_ref[...] = jnp.dot(zd2_ref[...], wa_ref[...],
                           preferred_element_type=jnp.float32)
    bm2_ref[...] = jnp.dot(zm2_ref[...], wb_ref[...],
                           preferred_element_type=jnp.float32) + b1t_ref[...]


def _precompute(zd2, zm2, wa, wb, b1t):
    return pl.pallas_call(
        _precompute_body,
        out_shape=[
            jax.ShapeDtypeStruct((_PROWS, 128), jnp.float32),
            jax.ShapeDtypeStruct((_PROWS, 128), jnp.float32),
        ],
    )(zd2, zm2, wa, wb, b1t)


def _edge_body(ad_hbm, bm_hbm, eli_hbm, aux_hbm, out_hbm,
               r0, c0, a0, b0, o0, r1, c1, a1, b1_, o1, aux_v,
               gs0, gs1, is0, is1, os0, os1):
    wid = lax.axis_index("s") * _NC + lax.axis_index("c")
    tbase = wid * _EPW
    B0 = (r0, c0, a0, b0, o0, gs0, is0, os0)
    B1 = (r1, c1, a1, b1_, o1, gs1, is1, os1)

    pltpu.sync_copy(aux_hbm, aux_v)
    iota = lax.iota(jnp.int32, 16)
    w2f = [plsc.load_gather(aux_v, [jnp.full((16,), 1 + f, jnp.int32)])
           for f in range(_HID)]
    vb2 = plsc.load_gather(aux_v, [jnp.full((16,), 1 + _HID, jnp.int32)])

    def idx_issue(buf, i):
        r, c, _, _, _, _, isem, _ = buf
        gbase = tbase + i * _C
        pltpu.async_copy(eli_hbm.at[0, pl.ds(gbase, _C)], r, isem)
        pltpu.async_copy(eli_hbm.at[1, pl.ds(gbase, _C)], c, isem)

    def idx_wait(buf):
        r, c, _, _, _, _, isem, _ = buf
        pltpu.make_async_copy(eli_hbm.at[0, pl.ds(0, _C)], r, isem).wait()
        pltpu.make_async_copy(eli_hbm.at[1, pl.ds(0, _C)], c, isem).wait()

    def fire(buf):
        r, c, za, zb, _, gsem, _, _ = buf
        for j in range(_NG):
            sl = pl.ds(j * _G, _G)
            pltpu.async_copy(ad_hbm.at[r.at[sl]], za.at[sl], gsem)
            pltpu.async_copy(bm_hbm.at[c.at[sl]], zb.at[sl], gsem)

    def drain(buf):
        _, _, za, zb, _, gsem, _, _ = buf
        pltpu.make_async_copy(ad_hbm.at[pl.ds(0, _C)], za, gsem).wait()
        pltpu.make_async_copy(bm_hbm.at[pl.ds(0, _C)], zb, gsem).wait()

    def compute_store(buf, i):
        _, _, za, zb, o, _, _, osem = buf
        gbase = tbase + i * _C

        @plsc.parallel_loop(0, _NGRP, 1, unroll=_UNROLL)
        def grp_body(g):
            base = g * 16
            lanes = base + iota
            acc = vb2
            for f in range(_HID):
                fi = jnp.full((16,), f, jnp.int32)
                a = plsc.load_gather(za, [lanes, fi])
                b = plsc.load_gather(zb, [lanes, fi])
                acc = acc + w2f[f] * jnp.maximum(a + b, 0.0)
            o[pl.ds(base, 16)] = acc

        pltpu.async_copy(o, out_hbm.at[pl.ds(gbase, _C)], osem)

    def out_wait(buf):
        _, _, _, _, o, _, _, osem = buf
        pltpu.make_async_copy(o, out_hbm.at[pl.ds(0, _C)], osem).wait()

    idx_issue(B0, 0)
    idx_wait(B0)
    fire(B0)
    idx_issue(B1, 1)

    idx_wait(B1); fire(B1)
    drain(B0); idx_issue(B0, 2)
    compute_store(B0, 0)
    idx_wait(B0); fire(B0)
    drain(B1); idx_issue(B1, 3)
    compute_store(B1, 1)

    def pair_body(p, carry):
        i = 2 * p
        idx_wait(B1); fire(B1)
        drain(B0); idx_issue(B0, i + 2)
        out_wait(B0)
        compute_store(B0, i)
        idx_wait(B0); fire(B0)
        drain(B1); idx_issue(B1, i + 3)
        out_wait(B1)
        compute_store(B1, i + 1)
        return carry

    lax.fori_loop(1, _NCHUNK // 2 - 1, pair_body, 0)

    i = _NCHUNK - 2
    idx_wait(B1); fire(B1)
    drain(B0)
    out_wait(B0)
    compute_store(B0, i)
    drain(B1)
    out_wait(B1)
    compute_store(B1, i + 1)
    out_wait(B0)
    out_wait(B1)


@functools.partial(jax.jit, static_argnums=())
def kernel(z_demand, z_measurement, edge_label_index, W1, b1, W2, b2):
    eye16 = jnp.eye(16, dtype=jnp.float32)
    wa = jnp.kron(eye16, W1[:, :_HID].T)
    wb = jnp.kron(eye16, W1[:, _HID:].T)
    b1t = jnp.tile(b1, 16).reshape(1, 128)
    ad2, bm2 = _precompute(z_demand.reshape(_PROWS, 128),
                           z_measurement.reshape(_PROWS, 128), wa, wb, b1t)
    ad = ad2.reshape(_N_NODES, _HID)
    bm = bm2.reshape(_N_NODES, _HID)

    eli = edge_label_index
    if eli.dtype != jnp.int32:
        eli = eli.astype(jnp.int32)
    aux = jnp.concatenate([jnp.zeros((1,), jnp.float32), W2.reshape(-1),
                           b2.reshape(-1),
                           jnp.zeros((16 - _HID - 2,), jnp.float32)])

    edge_kernel = pl.kernel(
        _edge_body,
        out_type=jax.ShapeDtypeStruct((_N_EDGES,), jnp.float32),
        mesh=plsc.VectorSubcoreMesh(core_axis_name="c", subcore_axis_name="s"),
        compiler_params=pltpu.CompilerParams(needs_layout_passes=False,
                                             use_tc_tiling_on_sc=False),
        scratch_types=[
            pltpu.VMEM((_C,), jnp.int32),
            pltpu.VMEM((_C,), jnp.int32),
            pltpu.VMEM((_C, _HID), jnp.float32),
            pltpu.VMEM((_C, _HID), jnp.float32),
            pltpu.VMEM((_C,), jnp.float32),
            pltpu.VMEM((_C,), jnp.int32),
            pltpu.VMEM((_C,), jnp.int32),
            pltpu.VMEM((_C, _HID), jnp.float32),
            pltpu.VMEM((_C, _HID), jnp.float32),
            pltpu.VMEM((_C,), jnp.float32),
            pltpu.VMEM((16,), jnp.float32),
            pltpu.SemaphoreType.DMA,
            pltpu.SemaphoreType.DMA,
            pltpu.SemaphoreType.DMA,
            pltpu.SemaphoreType.DMA,
            pltpu.SemaphoreType.DMA,
            pltpu.SemaphoreType.DMA,
        ],
    )
    return edge_kernel(ad, bm, eli, aux)
